# baseline (speedup 1.0000x reference)
import functools

import jax
import jax.numpy as jnp
from jax import lax
from jax.experimental import pallas as pl
from jax.experimental.pallas import tpu as pltpu
from jax.experimental.pallas import tpu_sc as plsc

VOCAB = 1_000_000
D = 64
DP = 128
B = 4096
T = 200

NC = 2
NS = 16
NW = NC * NS
RPW = B // NW

W = 512

CH0, CH1 = 128, 72

L = 16
NV = D // L
SCALE = 1.0 / T


def _xpose_body(x_ref, o_ref):
  x = x_ref[...]
  r = lax.broadcasted_iota(jnp.int32, (D, D), 0)
  c = lax.broadcasted_iota(jnp.int32, (D, D), 1)
  eye = (r == c).astype(jnp.float32)
  xt = lax.dot_general(x, eye, (((0,), (0,)), ((), ())),
                       preferred_element_type=jnp.float32)
  o_ref[...] = jnp.concatenate([xt, xt], axis=1)


def _gather_body(tok_hbm, table_hbm, out_hbm, idx_v, buf_v, out_v, sem):
  wid = lax.axis_index("s") * NC + lax.axis_index("c")
  base = wid * RPW

  pltpu.sync_copy(tok_hbm.at[pl.ds(base, RPW)], idx_v)

  def issue(r, sl):
    pltpu.async_copy(
        table_hbm.at[idx_v.at[r, pl.ds(0, CH0)]],
        buf_v.at[sl, pl.ds(0, CH0)], sem)
    pltpu.async_copy(
        table_hbm.at[idx_v.at[r, pl.ds(CH0, CH1)]],
        buf_v.at[sl, pl.ds(CH0, CH1)], sem)

  def drain(r, sl):
    pltpu.make_async_copy(
        table_hbm.at[idx_v.at[r, pl.ds(0, CH0)]],
        buf_v.at[sl, pl.ds(0, CH0)], sem).wait()
    pltpu.make_async_copy(
        table_hbm.at[idx_v.at[r, pl.ds(CH0, CH1)]],
        buf_v.at[sl, pl.ds(CH0, CH1)], sem).wait()

  issue(0, 0)

  def do_row(r, _):
    sl = lax.rem(r, 2)
    drain(r, sl)

    @pl.when(r + 1 < RPW)
    def _prefetch():
      issue(r + 1, 1 - sl)

    def acc_grp(tg, accs):
      upd = accs
      for dt in range(8):
        t = tg * 8 + dt
        upd = tuple(upd[c] + buf_v[sl, t, pl.ds(c * L, L)] for c in range(NV))
      return upd

    zeros = tuple(jnp.zeros((L,), jnp.float32) for _ in range(NV))
    accs = lax.fori_loop(0, T // 8, acc_grp, zeros)
    for c in range(NV):
      out_v[r, pl.ds(c * L, L)] = accs[c] * SCALE
    return ()

  lax.fori_loop(0, RPW, do_row, ())

  pltpu.sync_copy(out_v, out_hbm.at[pl.ds(base, RPW)])


@functools.partial(jax.jit, static_argnames=())
def _encoder(token_ids, table):
  nblk = (VOCAB + W - 1) // W
  xpose = pl.pallas_call(
      _xpose_body,
      grid=(nblk,),
      in_specs=[pl.BlockSpec((D, W), lambda i: (0, i))],
      out_specs=pl.BlockSpec((W, DP), lambda i: (i, 0)),
      out_shape=jax.ShapeDtypeStruct((VOCAB, DP), jnp.float32),
      compiler_params=pltpu.CompilerParams(
          dimension_semantics=("arbitrary",)),
  )
  table_rm = xpose(table.T)

  mesh = plsc.VectorSubcoreMesh(
      core_axis_name="c", subcore_axis_name="s", num_cores=NC,
      num_subcores=NS)
  gather = pl.kernel(
      _gather_body,
      out_type=jax.ShapeDtypeStruct((B, D), jnp.float32),
      mesh=mesh,
      scratch_types=[
          pltpu.VMEM((RPW, T), jnp.int32),
          pltpu.VMEM((2, T, DP), jnp.float32),
          pltpu.VMEM((RPW, D), jnp.float32),
          pltpu.SemaphoreType.DMA,
      ],
  )
  return gather(token_ids, table_rm)


def kernel(token_ids, table):
  return _encoder(token_ids.astype(jnp.int32), table)


# speedup vs baseline: 2.0623x; 2.0623x over previous
import functools

import jax
import jax.numpy as jnp
from jax import lax
from jax.experimental import pallas as pl
from jax.experimental.pallas import tpu as pltpu
from jax.experimental.pallas import tpu_sc as plsc

VOCAB = 1_000_000
D = 64
DP = 128
B = 4096
T = 200

NC = 2
NS = 16
NW = NC * NS
RPW = B // NW

W = 2048

CH0, CH1 = 128, 72

L = 16
NV = D // L
SCALE = 1.0 / T


def _xpose_body(x_ref, o_ref):
  x = x_ref[...]
  r = lax.broadcasted_iota(jnp.int32, (D, D), 0)
  c = lax.broadcasted_iota(jnp.int32, (D, D), 1)
  eye = (r == c).astype(jnp.float32)
  xt = lax.dot_general(x, eye, (((0,), (0,)), ((), ())),
                       preferred_element_type=jnp.float32)
  o_ref[...] = jnp.concatenate([xt, xt], axis=1)


def _gather_body(tok_hbm, table_hbm, out_hbm, idx_v, buf_v, out_v, sem):
  wid = lax.axis_index("s") * NC + lax.axis_index("c")
  base = wid * RPW

  pltpu.sync_copy(tok_hbm.at[pl.ds(base, RPW)], idx_v)

  def issue(r, sl):
    pltpu.async_copy(
        table_hbm.at[idx_v.at[r, pl.ds(0, CH0)]],
        buf_v.at[sl, pl.ds(0, CH0)], sem)
    pltpu.async_copy(
        table_hbm.at[idx_v.at[r, pl.ds(CH0, CH1)]],
        buf_v.at[sl, pl.ds(CH0, CH1)], sem)

  def drain(r, sl):
    pltpu.make_async_copy(
        table_hbm.at[idx_v.at[r, pl.ds(0, CH0)]],
        buf_v.at[sl, pl.ds(0, CH0)], sem).wait()
    pltpu.make_async_copy(
        table_hbm.at[idx_v.at[r, pl.ds(CH0, CH1)]],
        buf_v.at[sl, pl.ds(CH0, CH1)], sem).wait()

  issue(0, 0)

  def do_row(r, _):
    sl = lax.rem(r, 2)
    drain(r, sl)

    @pl.when(r + 1 < RPW)
    def _prefetch():
      issue(r + 1, 1 - sl)

    def acc_grp(tg, accs):
      upd = accs
      for dt in range(8):
        t = tg * 8 + dt
        upd = tuple(upd[c] + buf_v[sl, t, pl.ds(c * L, L)] for c in range(NV))
      return upd

    zeros = tuple(jnp.zeros((L,), jnp.float32) for _ in range(NV))
    accs = lax.fori_loop(0, T // 8, acc_grp, zeros)
    for c in range(NV):
      out_v[r, pl.ds(c * L, L)] = accs[c] * SCALE
    return ()

  lax.fori_loop(0, RPW, do_row, ())

  pltpu.sync_copy(out_v, out_hbm.at[pl.ds(base, RPW)])


@functools.partial(jax.jit, static_argnames=())
def _encoder(token_ids, table):
  nblk = (VOCAB + W - 1) // W
  xpose = pl.pallas_call(
      _xpose_body,
      grid=(nblk,),
      in_specs=[pl.BlockSpec((D, W), lambda i: (0, i))],
      out_specs=pl.BlockSpec((W, DP), lambda i: (i, 0)),
      out_shape=jax.ShapeDtypeStruct((VOCAB, DP), jnp.float32),
      compiler_params=pltpu.CompilerParams(
          dimension_semantics=("arbitrary",)),
  )
  table_rm = xpose(table.T)

  mesh = plsc.VectorSubcoreMesh(
      core_axis_name="c", subcore_axis_name="s", num_cores=NC,
      num_subcores=NS)
  gather = pl.kernel(
      _gather_body,
      out_type=jax.ShapeDtypeStruct((B, D), jnp.float32),
      mesh=mesh,
      scratch_types=[
          pltpu.VMEM((RPW, T), jnp.int32),
          pltpu.VMEM((2, T, DP), jnp.float32),
          pltpu.VMEM((RPW, D), jnp.float32),
          pltpu.SemaphoreType.DMA,
      ],
  )
  return gather(token_ids, table_rm)


def kernel(token_ids, table):
  return _encoder(token_ids.astype(jnp.int32), table)


# speedup vs baseline: 2.7768x; 1.3464x over previous
import functools

import jax
import jax.numpy as jnp
from jax import lax
from jax.experimental import pallas as pl
from jax.experimental.pallas import tpu as pltpu
from jax.experimental.pallas import tpu_sc as plsc

VOCAB = 1_000_000
D = 64
DP = 128
B = 4096
T = 200

NC = 2
NS = 16
NW = NC * NS
RPW = B // NW

W = 8192

CH0, CH1 = 128, 72

L = 16
NV = D // L
SCALE = 1.0 / T


def _xpose_body(x_ref, o_ref):
  x = x_ref[...]
  r = lax.broadcasted_iota(jnp.int32, (D, D), 0)
  c = lax.broadcasted_iota(jnp.int32, (D, D), 1)
  eye = (r == c).astype(jnp.float32)
  xt = lax.dot_general(x, eye, (((0,), (0,)), ((), ())),
                       preferred_element_type=jnp.float32)
  o_ref[...] = jnp.concatenate([xt, xt], axis=1)


def _gather_body(tok_hbm, table_hbm, out_hbm, idx_v, buf_v, out_v, sem):
  wid = lax.axis_index("s") * NC + lax.axis_index("c")
  base = wid * RPW

  pltpu.sync_copy(tok_hbm.at[pl.ds(base, RPW)], idx_v)

  def issue(r, sl):
    pltpu.async_copy(
        table_hbm.at[idx_v.at[r, pl.ds(0, CH0)]],
        buf_v.at[sl, pl.ds(0, CH0)], sem)
    pltpu.async_copy(
        table_hbm.at[idx_v.at[r, pl.ds(CH0, CH1)]],
        buf_v.at[sl, pl.ds(CH0, CH1)], sem)

  def drain(r, sl):
    pltpu.make_async_copy(
        table_hbm.at[idx_v.at[r, pl.ds(0, CH0)]],
        buf_v.at[sl, pl.ds(0, CH0)], sem).wait()
    pltpu.make_async_copy(
        table_hbm.at[idx_v.at[r, pl.ds(CH0, CH1)]],
        buf_v.at[sl, pl.ds(CH0, CH1)], sem).wait()

  issue(0, 0)

  def do_row(r, _):
    sl = lax.rem(r, 2)
    drain(r, sl)

    @pl.when(r + 1 < RPW)
    def _prefetch():
      issue(r + 1, 1 - sl)

    def acc_grp(tg, accs):
      upd = accs
      for dt in range(8):
        t = tg * 8 + dt
        upd = tuple(upd[c] + buf_v[sl, t, pl.ds(c * L, L)] for c in range(NV))
      return upd

    zeros = tuple(jnp.zeros((L,), jnp.float32) for _ in range(NV))
    accs = lax.fori_loop(0, T // 8, acc_grp, zeros)
    for c in range(NV):
      out_v[r, pl.ds(c * L, L)] = accs[c] * SCALE
    return ()

  lax.fori_loop(0, RPW, do_row, ())

  pltpu.sync_copy(out_v, out_hbm.at[pl.ds(base, RPW)])


@functools.partial(jax.jit, static_argnames=())
def _encoder(token_ids, table):
  nblk = (VOCAB + W - 1) // W
  xpose = pl.pallas_call(
      _xpose_body,
      grid=(nblk,),
      in_specs=[pl.BlockSpec((D, W), lambda i: (0, i))],
      out_specs=pl.BlockSpec((W, DP), lambda i: (i, 0)),
      out_shape=jax.ShapeDtypeStruct((VOCAB, DP), jnp.float32),
      compiler_params=pltpu.CompilerParams(
          dimension_semantics=("arbitrary",)),
  )
  table_rm = xpose(table.T)

  mesh = plsc.VectorSubcoreMesh(
      core_axis_name="c", subcore_axis_name="s", num_cores=NC,
      num_subcores=NS)
  gather = pl.kernel(
      _gather_body,
      out_type=jax.ShapeDtypeStruct((B, D), jnp.float32),
      mesh=mesh,
      scratch_types=[
          pltpu.VMEM((RPW, T), jnp.int32),
          pltpu.VMEM((2, T, DP), jnp.float32),
          pltpu.VMEM((RPW, D), jnp.float32),
          pltpu.SemaphoreType.DMA,
      ],
  )
  return gather(token_ids, table_rm)


def kernel(token_ids, table):
  return _encoder(token_ids.astype(jnp.int32), table)


# speedup vs baseline: 2.9423x; 1.0596x over previous
import functools

import jax
import jax.numpy as jnp
from jax import lax
from jax.experimental import pallas as pl
from jax.experimental.pallas import tpu as pltpu
from jax.experimental.pallas import tpu_sc as plsc

VOCAB = 1_000_000
D = 64
DP = 128
B = 4096
T = 200

NC = 2
NS = 16
NW = NC * NS
RPW = B // NW

W = 16384

CH0, CH1 = 128, 72

L = 16
NV = D // L
SCALE = 1.0 / T


def _xpose_body(x_ref, o_ref):
  x = x_ref[...]
  r = lax.broadcasted_iota(jnp.int32, (D, D), 0)
  c = lax.broadcasted_iota(jnp.int32, (D, D), 1)
  eye = (r == c).astype(jnp.float32)
  xt = lax.dot_general(x, eye, (((0,), (0,)), ((), ())),
                       preferred_element_type=jnp.float32)
  o_ref[...] = jnp.concatenate([xt, xt], axis=1)


def _gather_body(tok_hbm, table_hbm, out_hbm, idx_v, buf_v, out_v, sem):
  wid = lax.axis_index("s") * NC + lax.axis_index("c")
  base = wid * RPW

  pltpu.sync_copy(tok_hbm.at[pl.ds(base, RPW)], idx_v)

  def issue(r, sl):
    pltpu.async_copy(
        table_hbm.at[idx_v.at[r, pl.ds(0, CH0)]],
        buf_v.at[sl, pl.ds(0, CH0)], sem)
    pltpu.async_copy(
        table_hbm.at[idx_v.at[r, pl.ds(CH0, CH1)]],
        buf_v.at[sl, pl.ds(CH0, CH1)], sem)

  def drain(r, sl):
    pltpu.make_async_copy(
        table_hbm.at[idx_v.at[r, pl.ds(0, CH0)]],
        buf_v.at[sl, pl.ds(0, CH0)], sem).wait()
    pltpu.make_async_copy(
        table_hbm.at[idx_v.at[r, pl.ds(CH0, CH1)]],
        buf_v.at[sl, pl.ds(CH0, CH1)], sem).wait()

  issue(0, 0)

  def do_row(r, _):
    sl = lax.rem(r, 2)
    drain(r, sl)

    @pl.when(r + 1 < RPW)
    def _prefetch():
      issue(r + 1, 1 - sl)

    def acc_grp(tg, accs):
      upd = accs
      for dt in range(8):
        t = tg * 8 + dt
        upd = tuple(upd[c] + buf_v[sl, t, pl.ds(c * L, L)] for c in range(NV))
      return upd

    zeros = tuple(jnp.zeros((L,), jnp.float32) for _ in range(NV))
    accs = lax.fori_loop(0, T // 8, acc_grp, zeros)
    for c in range(NV):
      out_v[r, pl.ds(c * L, L)] = accs[c] * SCALE
    return ()

  lax.fori_loop(0, RPW, do_row, ())

  pltpu.sync_copy(out_v, out_hbm.at[pl.ds(base, RPW)])


@functools.partial(jax.jit, static_argnames=())
def _encoder(token_ids, table):
  nblk = (VOCAB + W - 1) // W
  xpose = pl.pallas_call(
      _xpose_body,
      grid=(nblk,),
      in_specs=[pl.BlockSpec((D, W), lambda i: (0, i))],
      out_specs=pl.BlockSpec((W, DP), lambda i: (i, 0)),
      out_shape=jax.ShapeDtypeStruct((VOCAB, DP), jnp.float32),
      compiler_params=pltpu.CompilerParams(
          dimension_semantics=("arbitrary",)),
  )
  table_rm = xpose(table.T)

  mesh = plsc.VectorSubcoreMesh(
      core_axis_name="c", subcore_axis_name="s", num_cores=NC,
      num_subcores=NS)
  gather = pl.kernel(
      _gather_body,
      out_type=jax.ShapeDtypeStruct((B, D), jnp.float32),
      mesh=mesh,
      scratch_types=[
          pltpu.VMEM((RPW, T), jnp.int32),
          pltpu.VMEM((2, T, DP), jnp.float32),
          pltpu.VMEM((RPW, D), jnp.float32),
          pltpu.SemaphoreType.DMA,
      ],
  )
  return gather(token_ids, table_rm)


def kernel(token_ids, table):
  return _encoder(token_ids.astype(jnp.int32), table)
